# baseline (speedup 1.0000x reference)
import jax
import jax.numpy as jnp
from jax import lax
from jax.experimental import pallas as pl
from jax.experimental.pallas import tpu as pltpu
from jax.experimental.pallas import tpu_sc as plsc

N = 100000
D = 18
DH = 9
DPH = 10
NC, NS = 2, 16
NP = 100352
NP10 = NP * DPH
RPT = NP // NS
ZRPT = NP10 // NS
EC = 128
K = 8
M = 98
E_PAD = NC * NS * K * M * EC
NCHUNK = E_PAD // EC
CPT = NCHUNK // (NC * NS)
CPT2 = NCHUNK // NS
NB = NP // 2048
BLK = 2048

_mesh = plsc.VectorSubcoreMesh(core_axis_name="c", subcore_axis_name="s")


def _deg_body(src_hbm, zeros_hbm, ones_hbm, deg, sidx, onesv, degsh):
    cid = lax.axis_index("c")
    sid = lax.axis_index("s")
    wid = cid * NS + sid
    r0 = sid * RPT
    pltpu.sync_copy(ones_hbm, onesv)
    pltpu.sync_copy(zeros_hbm, degsh.at[pl.ds(r0, RPT)])
    plsc.subcore_barrier()

    def outer(m, c):
        base = wid * CPT + m * K
        pltpu.sync_copy(src_hbm.at[pl.ds(base, K)], sidx)
        for j in range(K):
            pltpu.sync_copy(onesv, degsh.at[sidx.at[j]], add=True)
        return c

    lax.fori_loop(0, M, outer, 0)
    plsc.subcore_barrier()
    pltpu.sync_copy(degsh.at[pl.ds(r0, RPT)], deg.at[cid, pl.ds(r0, RPT)])


_deg_call = pl.kernel(
    _deg_body,
    out_type=jax.ShapeDtypeStruct((NC, NP), jnp.float32),
    mesh=_mesh,
    scratch_types=[
        pltpu.VMEM((K, EC), jnp.int32),
        pltpu.VMEM((EC,), jnp.float32),
        pltpu.VMEM_SHARED((NP,), jnp.float32),
    ],
)


def _agg_body(h_hbm, src_hbm, dst_hbm, zeros_hbm, out,
              sidx, didx, eidx, fidx, vals, aggsh, gsem, ssem):
    cid = lax.axis_index("c")
    sid = lax.axis_index("s")
    r0 = sid * ZRPT
    coff = cid * NP10
    pltpu.sync_copy(zeros_hbm, aggsh.at[pl.ds(r0, ZRPT)])
    plsc.subcore_barrier()

    def outer(m, c):
        chunk = sid * CPT2 + m
        pltpu.sync_copy(src_hbm.at[chunk], sidx)
        pltpu.sync_copy(dst_hbm.at[chunk], didx)
        for v in range(EC // 16):
            sb = sidx[pl.ds(v * 16, 16)] * DPH + coff
            db = didx[pl.ds(v * 16, 16)] * DPH
            for j in range(DPH):
                sl = pl.ds(j * EC + v * 16, 16)
                eidx[sl] = sb + j
                fidx[sl] = db + j
        pltpu.async_copy(h_hbm.at[eidx], vals, gsem).wait()
        pltpu.async_copy(vals, aggsh.at[fidx], ssem, add=True).wait()
        return c

    lax.fori_loop(0, CPT2, outer, 0)
    plsc.subcore_barrier()
    pltpu.sync_copy(aggsh.at[pl.ds(r0, ZRPT)], out.at[cid, pl.ds(r0, ZRPT)])


_agg_call = pl.kernel(
    _agg_body,
    out_type=jax.ShapeDtypeStruct((NC, NP10), jnp.float32),
    mesh=_mesh,
    scratch_types=[
        pltpu.VMEM((EC,), jnp.int32),
        pltpu.VMEM((EC,), jnp.int32),
        pltpu.VMEM((DPH * EC,), jnp.int32),
        pltpu.VMEM((DPH * EC,), jnp.int32),
        pltpu.VMEM((DPH * EC,), jnp.float32),
        pltpu.VMEM_SHARED((NP10,), jnp.float32),
        pltpu.SemaphoreType.DMA,
        pltpu.SemaphoreType.DMA,
    ],
)


def _scale_body(feat_ref, d0_ref, d1_ref, out_ref):
    deg = jnp.maximum(d0_ref[...] + d1_ref[...], 1.0)
    h = feat_ref[...] * lax.rsqrt(deg)
    ones = jnp.ones((BLK, 1), jnp.float32)
    zeros = jnp.zeros((BLK, 1), jnp.float32)
    lo = jnp.concatenate([h[:, :DH], ones], axis=1)
    hi = jnp.concatenate([h[:, DH:], zeros], axis=1)
    out_ref[...] = jnp.where(pl.program_id(0) >= NB, hi, lo)


_scale_call = pl.pallas_call(
    _scale_body,
    grid=(2 * NB,),
    in_specs=[
        pl.BlockSpec((BLK, D), lambda i: (i % NB, 0)),
        pl.BlockSpec((BLK, 1), lambda i: (i % NB, 0)),
        pl.BlockSpec((BLK, 1), lambda i: (i % NB, 0)),
    ],
    out_specs=pl.BlockSpec((BLK, DPH), lambda i: (i, 0)),
    out_shape=jax.ShapeDtypeStruct((2 * NP, DPH), jnp.float32),
)


def _dense_body(feat_ref, a0_ref, a1_ref, wc_ref, bc_ref, wl_ref, bl_ref,
                wt_ref, wb_ref, bo_ref, out_ref):
    a0 = a0_ref[...]
    indeg = jnp.maximum(a0[:, DH:DH + 1], 1.0)
    rst = (jnp.concatenate([a0[:, :DH], a1_ref[...][:, :DH]], axis=1)
           * lax.rsqrt(indeg))
    gf = jnp.maximum(
        jnp.dot(rst, wc_ref[...], preferred_element_type=jnp.float32)
        + bc_ref[...], 0.0)
    tra = jnp.dot(feat_ref[...], wl_ref[...],
                  preferred_element_type=jnp.float32) + bl_ref[...]
    out_ref[...] = (
        jnp.dot(jnp.maximum(tra, 0.0), wt_ref[...],
                preferred_element_type=jnp.float32)
        + jnp.dot(gf, wb_ref[...], preferred_element_type=jnp.float32)
        + bo_ref[...])


_dense_call = pl.pallas_call(
    _dense_body,
    grid=(NB,),
    in_specs=[
        pl.BlockSpec((BLK, D), lambda i: (i, 0)),
        pl.BlockSpec((BLK, DPH), lambda i: (i, 0)),
        pl.BlockSpec((BLK, DPH), lambda i: (i, 0)),
        pl.BlockSpec((D, D), lambda i: (0, 0)),
        pl.BlockSpec((1, D), lambda i: (0, 0)),
        pl.BlockSpec((D, D), lambda i: (0, 0)),
        pl.BlockSpec((1, D), lambda i: (0, 0)),
        pl.BlockSpec((D, 2), lambda i: (0, 0)),
        pl.BlockSpec((D, 2), lambda i: (0, 0)),
        pl.BlockSpec((1, 2), lambda i: (0, 0)),
    ],
    out_specs=pl.BlockSpec((BLK, 2), lambda i: (i, 0)),
    out_shape=jax.ShapeDtypeStruct((NP, 2), jnp.float32),
)


def kernel(feat, edge_index, W_conv, b_conv, W_lin, b_lin, W_cls, b_cls):
    src = edge_index[0]
    dst = edge_index[1]
    e = src.shape[0]
    trash = jnp.full((E_PAD - e,), N, jnp.int32)
    src2d = jnp.concatenate([src, trash]).reshape(NCHUNK, EC)
    dst2d = jnp.concatenate([dst, trash]).reshape(NCHUNK, EC)
    feat_pad = jnp.pad(feat, ((0, NP - N), (0, 0)))
    zeros1 = jnp.zeros((RPT,), jnp.float32)
    zeros2 = jnp.zeros((ZRPT,), jnp.float32)
    ones1 = jnp.ones((EC,), jnp.float32)

    deg = _deg_call(src2d, zeros1, ones1)
    h2 = _scale_call(feat_pad, deg[0].reshape(NP, 1), deg[1].reshape(NP, 1))
    agg = _agg_call(h2.reshape(2 * NP10), src2d, dst2d, zeros2)
    a0 = agg[0].reshape(NP, DPH)
    a1 = agg[1].reshape(NP, DPH)
    out = _dense_call(feat_pad, a0, a1, W_conv, b_conv.reshape(1, D),
                      W_lin, b_lin.reshape(1, D), W_cls[:D], W_cls[D:],
                      b_cls.reshape(1, 2))
    return out[:N]


# speedup vs baseline: 1.3626x; 1.3626x over previous
import jax
import jax.numpy as jnp
from jax import lax
from jax.experimental import pallas as pl
from jax.experimental.pallas import tpu as pltpu
from jax.experimental.pallas import tpu_sc as plsc

N = 100000
D = 18
DH = 9
DPH = 10
NC, NS = 2, 16
NP = 100352
NP10 = NP * DPH
RPT = NP // NS
ZRPT = NP10 // NS
EC = 128
K = 8
M = 98
E_PAD = NC * NS * K * M * EC
NCHUNK = E_PAD // EC
CPT = NCHUNK // (NC * NS)
CPT2 = NCHUNK // NS
KA = 4
NB = NP // 2048
BLK = 2048

_mesh = plsc.VectorSubcoreMesh(core_axis_name="c", subcore_axis_name="s")


def _deg_body(src_hbm, zeros_hbm, ones_hbm, deg, sidx, onesv, degsh):
    cid = lax.axis_index("c")
    sid = lax.axis_index("s")
    wid = cid * NS + sid
    r0 = sid * RPT
    pltpu.sync_copy(ones_hbm, onesv)
    pltpu.sync_copy(zeros_hbm, degsh.at[pl.ds(r0, RPT)])
    plsc.subcore_barrier()

    def outer(m, c):
        base = wid * CPT + m * K
        pltpu.sync_copy(src_hbm.at[pl.ds(base, K)], sidx)
        for j in range(K):
            pltpu.sync_copy(onesv, degsh.at[sidx.at[j]], add=True)
        return c

    lax.fori_loop(0, M, outer, 0)
    plsc.subcore_barrier()
    pltpu.sync_copy(degsh.at[pl.ds(r0, RPT)], deg.at[cid, pl.ds(r0, RPT)])


_deg_call = pl.kernel(
    _deg_body,
    out_type=jax.ShapeDtypeStruct((NC, NP), jnp.float32),
    mesh=_mesh,
    scratch_types=[
        pltpu.VMEM((K, EC), jnp.int32),
        pltpu.VMEM((EC,), jnp.float32),
        pltpu.VMEM_SHARED((NP,), jnp.float32),
    ],
)


def _agg_body(h_hbm, src_hbm, dst_hbm, zeros_hbm, out, *scr):
    sidx, didx = scr[0], scr[1]
    eidx = scr[2:2 + KA]
    fidx = scr[2 + KA:2 + 2 * KA]
    vals = scr[2 + 2 * KA:2 + 3 * KA]
    aggsh, gsem, ssem = scr[2 + 3 * KA:]
    cid = lax.axis_index("c")
    sid = lax.axis_index("s")
    r0 = sid * ZRPT
    coff = cid * NP10
    pltpu.sync_copy(zeros_hbm, aggsh.at[pl.ds(r0, ZRPT)])
    plsc.subcore_barrier()

    def outer(m, c):
        base = sid * CPT2 + m * KA
        pltpu.sync_copy(src_hbm.at[pl.ds(base, KA)], sidx)
        pltpu.sync_copy(dst_hbm.at[pl.ds(base, KA)], didx)
        for k in range(KA):
            for v in range(EC // 16):
                sb = sidx[k, pl.ds(v * 16, 16)] * DPH + coff
                db = didx[k, pl.ds(v * 16, 16)] * DPH
                for j in range(DPH):
                    sl = pl.ds(j * EC + v * 16, 16)
                    eidx[k][sl] = sb + j
                    fidx[k][sl] = db + j
        gcps = [pltpu.async_copy(h_hbm.at[eidx[k]], vals[k], gsem)
                for k in range(KA)]
        for cp in gcps:
            cp.wait()
        scps = [pltpu.async_copy(vals[k], aggsh.at[fidx[k]], ssem,
                                 add=True) for k in range(KA)]
        for cp in scps:
            cp.wait()
        return c

    lax.fori_loop(0, CPT2 // KA, outer, 0)
    plsc.subcore_barrier()
    pltpu.sync_copy(aggsh.at[pl.ds(r0, ZRPT)], out.at[cid, pl.ds(r0, ZRPT)])


_agg_call = pl.kernel(
    _agg_body,
    out_type=jax.ShapeDtypeStruct((NC, NP10), jnp.float32),
    mesh=_mesh,
    scratch_types=[
        pltpu.VMEM((KA, EC), jnp.int32),
        pltpu.VMEM((KA, EC), jnp.int32),
        *[pltpu.VMEM((DPH * EC,), jnp.int32) for _ in range(2 * KA)],
        *[pltpu.VMEM((DPH * EC,), jnp.float32) for _ in range(KA)],
        pltpu.VMEM_SHARED((NP10,), jnp.float32),
        pltpu.SemaphoreType.DMA,
        pltpu.SemaphoreType.DMA,
    ],
)


def _scale_body(feat_ref, d0_ref, d1_ref, out_ref):
    deg = jnp.maximum(d0_ref[...] + d1_ref[...], 1.0)
    h = feat_ref[...] * lax.rsqrt(deg)
    ones = jnp.ones((BLK, 1), jnp.float32)
    zeros = jnp.zeros((BLK, 1), jnp.float32)
    lo = jnp.concatenate([h[:, :DH], ones], axis=1)
    hi = jnp.concatenate([h[:, DH:], zeros], axis=1)
    out_ref[...] = jnp.where(pl.program_id(0) >= NB, hi, lo)


_scale_call = pl.pallas_call(
    _scale_body,
    grid=(2 * NB,),
    in_specs=[
        pl.BlockSpec((BLK, D), lambda i: (i % NB, 0)),
        pl.BlockSpec((BLK, 1), lambda i: (i % NB, 0)),
        pl.BlockSpec((BLK, 1), lambda i: (i % NB, 0)),
    ],
    out_specs=pl.BlockSpec((BLK, DPH), lambda i: (i, 0)),
    out_shape=jax.ShapeDtypeStruct((2 * NP, DPH), jnp.float32),
)


def _dense_body(feat_ref, a0_ref, a1_ref, wc_ref, bc_ref, wl_ref, bl_ref,
                wt_ref, wb_ref, bo_ref, out_ref):
    a0 = a0_ref[...]
    indeg = jnp.maximum(a0[:, DH:DH + 1], 1.0)
    rst = (jnp.concatenate([a0[:, :DH], a1_ref[...][:, :DH]], axis=1)
           * lax.rsqrt(indeg))
    gf = jnp.maximum(
        jnp.dot(rst, wc_ref[...], preferred_element_type=jnp.float32)
        + bc_ref[...], 0.0)
    tra = jnp.dot(feat_ref[...], wl_ref[...],
                  preferred_element_type=jnp.float32) + bl_ref[...]
    out_ref[...] = (
        jnp.dot(jnp.maximum(tra, 0.0), wt_ref[...],
                preferred_element_type=jnp.float32)
        + jnp.dot(gf, wb_ref[...], preferred_element_type=jnp.float32)
        + bo_ref[...])


_dense_call = pl.pallas_call(
    _dense_body,
    grid=(NB,),
    in_specs=[
        pl.BlockSpec((BLK, D), lambda i: (i, 0)),
        pl.BlockSpec((BLK, DPH), lambda i: (i, 0)),
        pl.BlockSpec((BLK, DPH), lambda i: (i, 0)),
        pl.BlockSpec((D, D), lambda i: (0, 0)),
        pl.BlockSpec((1, D), lambda i: (0, 0)),
        pl.BlockSpec((D, D), lambda i: (0, 0)),
        pl.BlockSpec((1, D), lambda i: (0, 0)),
        pl.BlockSpec((D, 2), lambda i: (0, 0)),
        pl.BlockSpec((D, 2), lambda i: (0, 0)),
        pl.BlockSpec((1, 2), lambda i: (0, 0)),
    ],
    out_specs=pl.BlockSpec((BLK, 2), lambda i: (i, 0)),
    out_shape=jax.ShapeDtypeStruct((NP, 2), jnp.float32),
)


def kernel(feat, edge_index, W_conv, b_conv, W_lin, b_lin, W_cls, b_cls):
    src = edge_index[0]
    dst = edge_index[1]
    e = src.shape[0]
    trash = jnp.full((E_PAD - e,), N, jnp.int32)
    src2d = jnp.concatenate([src, trash]).reshape(NCHUNK, EC)
    dst2d = jnp.concatenate([dst, trash]).reshape(NCHUNK, EC)
    feat_pad = jnp.pad(feat, ((0, NP - N), (0, 0)))
    zeros1 = jnp.zeros((RPT,), jnp.float32)
    zeros2 = jnp.zeros((ZRPT,), jnp.float32)
    ones1 = jnp.ones((EC,), jnp.float32)

    deg = _deg_call(src2d, zeros1, ones1)
    h2 = _scale_call(feat_pad, deg[0].reshape(NP, 1), deg[1].reshape(NP, 1))
    agg = _agg_call(h2.reshape(2 * NP10), src2d, dst2d, zeros2)
    a0 = agg[0].reshape(NP, DPH)
    a1 = agg[1].reshape(NP, DPH)
    out = _dense_call(feat_pad, a0, a1, W_conv, b_conv.reshape(1, D),
                      W_lin, b_lin.reshape(1, D), W_cls[:D], W_cls[D:],
                      b_cls.reshape(1, 2))
    return out[:N]
